# baseline (speedup 1.0000x reference)
import functools

import jax
import jax.numpy as jnp
from jax import lax
from jax.experimental import pallas as pl
from jax.experimental.pallas import tpu as pltpu
from jax.experimental.pallas import tpu_sc as plsc

N_NODES = 98304
B = 16
NUM_NODE = 6144
E = 1572864
ETOT = E + N_NODES
COORD = 3
NODE_DIM = 48
EDGE_DIM = 16
GLOB_DIM = 32
EPS = 1e-5

EPB = E // B
CNT_E = EPB + NUM_NODE

NC, NS = 2, 16
NW = NC * NS
HALF_MAIN = EPB // 2
HALF_SELF = NUM_NODE // 2
CHUNK = 1024
KSUB = CHUNK // 128
MAIN_CHUNKS = HALF_MAIN // CHUNK
SELF_CHUNKS = HALF_SELF // CHUNK
NODES_PW = N_NODES // NW
EMB_CHUNKS = NODES_PW // CHUNK

MROWS = ETOT * EDGE_DIM // 128
EA_LANES = 8 * GLOB_DIM


@functools.cache
def _mesh():
    return plsc.VectorSubcoreMesh(
        core_axis_name="c", subcore_axis_name="s", num_cores=NC, num_subcores=NS)


@functools.cache
def _build_emb_kernel():
  @functools.partial(
      pl.kernel,
      out_type=jax.ShapeDtypeStruct((N_NODES, NODE_DIM), jnp.float32),
      mesh=_mesh(),
      compiler_params=pltpu.CompilerParams(use_tc_tiling_on_sc=False),
      scratch_types=[
          pltpu.VMEM((KSUB, 128), jnp.int32),
          pltpu.VMEM((CHUNK, NODE_DIM), jnp.float32),
          pltpu.VMEM((CHUNK, NODE_DIM), jnp.float32),
          pltpu.SemaphoreType.DMA,
          pltpu.SemaphoreType.DMA,
      ],
  )
  def _emb_kernel(cid_hbm, tbl_hbm, eout_hbm, idc, ebuf0, ebuf1, gsem, wsem):
    w = lax.axis_index("s") * NC + lax.axis_index("c")
    base128 = w * (NODES_PW // 128)
    bufs = (ebuf0, ebuf1)

    def gather_chunk(cidx, buf):
        b128 = base128 + cidx * KSUB
        pltpu.sync_copy(cid_hbm.at[pl.ds(b128, KSUB)], idc)
        return [pltpu.async_copy(tbl_hbm.at[idc.at[j]],
                                 buf.at[pl.ds(j * 128, 128)], gsem)
                for j in range(KSUB)]

    cps = gather_chunk(0, bufs[0])
    for cidx in range(EMB_CHUNKS):
        for cp in cps:
            cp.wait()
        if cidx + 1 < EMB_CHUNKS:
            nxt = gather_chunk(cidx + 1, bufs[(cidx + 1) % 2])
        else:
            nxt = []
        b128 = base128 + cidx * KSUB
        pltpu.async_copy(bufs[cidx % 2],
                         eout_hbm.at[pl.ds(b128 * 128, CHUNK)], wsem).wait()
        cps = nxt

  return _emb_kernel


def _node_body(x_ref, emb_ref, W1_ref, b1_ref, W2_ref, b2_ref,
               Wd_ref, bd_ref, wnc_ref, bnc_ref, wne_ref, bne_ref,
               We_ref, be_ref, h_ref, he_ref):
    dot = functools.partial(jnp.dot, preferred_element_type=jnp.float32)
    xb = x_ref[...]
    n1 = float(NUM_NODE * COORD)
    mean1 = jnp.sum(xb) / n1
    xc = xb - mean1
    var1 = jnp.sum(xc * xc) / n1
    hb = xc * lax.rsqrt(var1 + EPS) * wnc_ref[...] + bnc_ref[...]
    h1 = jnp.maximum(dot(hb, W1_ref[...]) + b1_ref[...], 0.0)
    h2 = jnp.maximum(dot(h1, W2_ref[...]) + b2_ref[...], 0.0)
    h3 = jnp.maximum(dot(h2, Wd_ref[...]) + bd_ref[...], 0.0)
    t = h3 + emb_ref[...]
    n2 = float(NUM_NODE * NODE_DIM)
    mean2 = jnp.sum(t) / n2
    tc2 = t - mean2
    var2 = jnp.sum(tc2 * tc2) / n2
    hn = tc2 * lax.rsqrt(var2 + EPS) * wne_ref[...] + bne_ref[...]
    h_ref[...] = hn
    he_ref[...] = jnp.maximum(dot(hn, We_ref[...]) + be_ref[...], 0.0)


def _node_stage(x, emb, W1, b1, W2, b2, Wd, bd, w_nc, b_nc,
                w_ne, b_ne, We, be):
    def full(a):
        return pl.BlockSpec(a.shape, lambda b_: tuple(0 for _ in a.shape))

    grid_specs = [
        pl.BlockSpec((NUM_NODE, COORD), lambda b_: (b_, 0)),
        pl.BlockSpec((NUM_NODE, NODE_DIM), lambda b_: (b_, 0)),
    ] + [full(a) for a in (W1, b1, W2, b2, Wd, bd, w_nc, b_nc, w_ne, b_ne, We, be)]
    return pl.pallas_call(
        _node_body,
        grid=(B,),
        in_specs=grid_specs,
        out_specs=[
            pl.BlockSpec((NUM_NODE, NODE_DIM), lambda b_: (b_, 0)),
            pl.BlockSpec((NUM_NODE, EDGE_DIM), lambda b_: (b_, 0)),
        ],
        out_shape=[
            jax.ShapeDtypeStruct((N_NODES, NODE_DIM), jnp.float32),
            jax.ShapeDtypeStruct((N_NODES, EDGE_DIM), jnp.float32),
        ],
    )(x, emb, W1, b1, W2, b2, Wd, bd, w_nc, b_nc, w_ne, b_ne, We, be)


@functools.cache
def _build_edge_kernel():
  @functools.partial(
      pl.kernel,
      out_type=(jax.ShapeDtypeStruct((MROWS, 128), jnp.float32),
                jax.ShapeDtypeStruct((2 * NW, EDGE_DIM), jnp.float32)),
      mesh=_mesh(),
      compiler_params=pltpu.CompilerParams(use_tc_tiling_on_sc=False),
      scratch_types=[
          pltpu.VMEM((KSUB, 128), jnp.int32),
          pltpu.VMEM((KSUB, 128), jnp.int32),
          pltpu.VMEM((CHUNK, EDGE_DIM), jnp.float32),
          pltpu.VMEM((CHUNK, EDGE_DIM), jnp.float32),
          pltpu.VMEM((CHUNK // 8, 128), jnp.float32),
          pltpu.VMEM((2, EDGE_DIM), jnp.float32),
          pltpu.SemaphoreType.DMA,
          pltpu.SemaphoreType.DMA,
      ],
  )
  def _edge_kernel(row_hbm, col_hbm, he_hbm, m_hbm, stats_hbm,
                   idxr, idxc, rbuf, cbuf, wbuf, sbuf, sem1, sem2):
    w = lax.axis_index("s") * NC + lax.axis_index("c")
    g = w // 2
    hf = w % 2
    main128 = g * (EPB // 128) + hf * (HALF_MAIN // 128)
    self128 = (E // 128) + g * (NUM_NODE // 128) + hf * (HALF_SELF // 128)

    def process(b128, acc):
        pltpu.sync_copy(row_hbm.at[pl.ds(b128, KSUB)], idxr)
        pltpu.sync_copy(col_hbm.at[pl.ds(b128, KSUB)], idxc)
        cps = [pltpu.async_copy(he_hbm.at[idxr.at[j]],
                                rbuf.at[pl.ds(j * 128, 128)], sem1)
               for j in range(KSUB)]
        cps += [pltpu.async_copy(he_hbm.at[idxc.at[j]],
                                 cbuf.at[pl.ds(j * 128, 128)], sem2)
                for j in range(KSUB)]
        for cp in cps:
            cp.wait()

        def body8(k, c2):
            s0, q0, s1, q1 = c2
            e0 = k * 8
            for u in range(8):
                m = (rbuf[e0 + u] + cbuf[e0 + u]) * 0.5
                wbuf[k, pl.ds(u * 16, 16)] = m
                if u % 2 == 0:
                    s0 = s0 + m
                    q0 = q0 + m * m
                else:
                    s1 = s1 + m
                    q1 = q1 + m * m
            return (s0, q0, s1, q1)

        acc = lax.fori_loop(0, CHUNK // 8, body8, acc)
        pltpu.sync_copy(wbuf, m_hbm.at[pl.ds(b128 * 16, CHUNK // 8)])
        return acc

    zero = jnp.zeros((16,), jnp.float32)
    acc = lax.fori_loop(
        0, MAIN_CHUNKS, lambda k, a: process(main128 + k * KSUB, a),
        (zero, zero, zero, zero))
    acc = lax.fori_loop(
        0, SELF_CHUNKS, lambda k, a: process(self128 + k * KSUB, a), acc)
    sbuf[0] = acc[0] + acc[2]
    sbuf[1] = acc[1] + acc[3]
    pltpu.sync_copy(sbuf.at[0], stats_hbm.at[w])
    pltpu.sync_copy(sbuf.at[1], stats_hbm.at[NW + w])

  return _edge_kernel


def _final_body(m_ref, stats_ref, Wg128_ref, bg256_ref, wen128_ref, ben128_ref,
                wgn_ref, bgn_ref, o0, o1, o2, o3, o4, o5, o6, o7, u_ref):
    b = pl.program_id(0)
    c = pl.program_id(1)
    stats = stats_ref[...]
    rid = lax.broadcasted_iota(jnp.int32, (2 * NW, EDGE_DIM), 0)
    sel_s = (rid // 2 == b) & (rid < NW)
    sel_q = (rid >= NW) & ((rid - NW) // 2 == b)
    S = jnp.sum(jnp.where(sel_s, stats, 0.0))
    Q = jnp.sum(jnp.where(sel_q, stats, 0.0))
    nrm = float(CNT_E * EDGE_DIM)
    mean = S / nrm
    var = Q / nrm - mean * mean
    inv = lax.rsqrt(var + EPS)
    ea = (m_ref[...] - mean) * inv * wen128_ref[...] + ben128_ref[...]
    for k, o_ref in enumerate((o0, o1, o2, o3, o4, o5, o6, o7)):
        o_ref[...] = ea[:, k * EDGE_DIM:(k + 1) * EDGE_DIM]
    g = jnp.maximum(
        jnp.dot(ea, Wg128_ref[...], preferred_element_type=jnp.float32)
        + bg256_ref[...], 0.0)
    psum = jnp.sum(g, axis=0, keepdims=True)[None]

    @pl.when(c == 0)
    def _():
        u_ref[...] = psum

    @pl.when(c != 0)
    def _():
        u_ref[...] = u_ref[...] + psum

    @pl.when(c == B)
    def _():
        acc = u_ref[...] / float(CNT_E)
        tot = acc[:, :, 0:GLOB_DIM]
        for k in range(1, 8):
            tot = tot + acc[:, :, k * GLOB_DIM:(k + 1) * GLOB_DIM]
        mu = jnp.sum(tot) / float(GLOB_DIM)
        d = tot - mu
        varu = jnp.sum(d * d) / float(GLOB_DIM)
        fin = (d * lax.rsqrt(varu + EPS) * wgn_ref[...][None]
               + bgn_ref[...][None])
        u_ref[...] = jnp.concatenate(
            [fin, jnp.zeros((1, 1, EA_LANES - GLOB_DIM), jnp.float32)], axis=-1)


def _final_stage(m, stats, Wg128, bg256, wen128, ben128, w_gn, b_gn):
    def full(a):
        return pl.BlockSpec(a.shape, lambda b_, c_: tuple(0 for _ in a.shape))

    def edge_map(b_, c_):
        return (jnp.where(c_ < B, b_ * B + c_, B * B + b_), 0)

    rows_per_block = NUM_NODE * EDGE_DIM // 128
    return pl.pallas_call(
        _final_body,
        grid=(B, B + 1),
        in_specs=[pl.BlockSpec((rows_per_block, 128), edge_map),
                  full(stats), full(Wg128), full(bg256), full(wen128),
                  full(ben128), full(w_gn), full(b_gn)],
        out_specs=[pl.BlockSpec((rows_per_block, EDGE_DIM), edge_map)] * 8 + [
            pl.BlockSpec((1, 1, EA_LANES), lambda b_, c_: (b_, 0, 0)),
        ],
        out_shape=[jax.ShapeDtypeStruct((MROWS, EDGE_DIM), jnp.float32)] * 8 + [
            jax.ShapeDtypeStruct((B, 1, EA_LANES), jnp.float32),
        ],
    )(m, stats, Wg128, bg256, wen128, ben128, w_gn, b_gn)


def kernel(x, atom_ids, aa_ids, edge_index, W1, b1, W2, b2, Wd, bd,
           atom_emb, aa_emb, w_nc, b_nc, w_ne, b_ne, We, be, w_en, b_en,
           Wg, bg, w_gn, b_gn):
    loops = jnp.arange(N_NODES, dtype=edge_index.dtype)
    row = jnp.concatenate([edge_index[0], loops])
    col = jnp.concatenate([edge_index[1], loops])
    ei = jnp.stack([row, col])
    row128 = row.astype(jnp.int32).reshape(ETOT // 128, 128)
    col128 = col.astype(jnp.int32).reshape(ETOT // 128, 128)
    cid128 = (atom_ids.astype(jnp.int32) * 32
              + aa_ids.astype(jnp.int32)).reshape(N_NODES // 128, 128)
    comb_tbl = (atom_emb.astype(jnp.float32)[:, None, :]
                + aa_emb.astype(jnp.float32)[None, :, :]).reshape(-1, NODE_DIM)

    r2 = lambda a: a.reshape(1, -1).astype(jnp.float32)
    emb = _build_emb_kernel()(cid128, comb_tbl)
    h, he = _node_stage(x, emb, W1, r2(b1), W2, r2(b2), Wd, r2(bd),
                        r2(w_nc), r2(b_nc), r2(w_ne), r2(b_ne), We, r2(be))
    m, stats = _build_edge_kernel()(row128, col128, he)
    Wg128 = jnp.kron(jnp.eye(8, dtype=jnp.float32), Wg.astype(jnp.float32))
    *oks, u3 = _final_stage(m, stats, Wg128, r2(jnp.tile(bg, 8)),
                            r2(jnp.tile(w_en, 8)), r2(jnp.tile(b_en, 8)),
                            r2(w_gn), r2(b_gn))
    edge_attr = jnp.stack(oks, axis=1).reshape(ETOT, EDGE_DIM)
    return (h, edge_attr, u3[:, 0, :GLOB_DIM], ei)


# speedup vs baseline: 1.2950x; 1.2950x over previous
import functools

import jax
import jax.numpy as jnp
from jax import lax
from jax.experimental import pallas as pl
from jax.experimental.pallas import tpu as pltpu
from jax.experimental.pallas import tpu_sc as plsc

N_NODES = 98304
B = 16
NUM_NODE = 6144
E = 1572864
ETOT = E + N_NODES
COORD = 3
NODE_DIM = 48
EDGE_DIM = 16
GLOB_DIM = 32
EPS = 1e-5

EPB = E // B
CNT_E = EPB + NUM_NODE

NC, NS = 2, 16
NW = NC * NS
HALF_MAIN = EPB // 2
HALF_SELF = NUM_NODE // 2
CHUNK = 1024
KSUB = CHUNK // 128
MAIN_CHUNKS = HALF_MAIN // CHUNK
SELF_CHUNKS = HALF_SELF // CHUNK
NODES_PW = N_NODES // NW
EMB_CHUNKS = NODES_PW // CHUNK

MROWS = ETOT * EDGE_DIM // 128
EA_LANES = 8 * GLOB_DIM


@functools.cache
def _mesh():
    return plsc.VectorSubcoreMesh(
        core_axis_name="c", subcore_axis_name="s", num_cores=NC, num_subcores=NS)


@functools.cache
def _build_emb_kernel():
  @functools.partial(
      pl.kernel,
      out_type=jax.ShapeDtypeStruct((N_NODES, NODE_DIM), jnp.float32),
      mesh=_mesh(),
      compiler_params=pltpu.CompilerParams(use_tc_tiling_on_sc=False),
      scratch_types=[
          pltpu.VMEM((KSUB, 128), jnp.int32),
          pltpu.VMEM((CHUNK, NODE_DIM), jnp.float32),
          pltpu.VMEM((CHUNK, NODE_DIM), jnp.float32),
          pltpu.SemaphoreType.DMA,
          pltpu.SemaphoreType.DMA,
      ],
  )
  def _emb_kernel(cid_hbm, tbl_hbm, eout_hbm, idc, ebuf0, ebuf1, gsem, wsem):
    w = lax.axis_index("s") * NC + lax.axis_index("c")
    base128 = w * (NODES_PW // 128)
    bufs = (ebuf0, ebuf1)

    def gather_chunk(cidx, buf):
        b128 = base128 + cidx * KSUB
        pltpu.sync_copy(cid_hbm.at[pl.ds(b128, KSUB)], idc)
        return [pltpu.async_copy(tbl_hbm.at[idc.at[j]],
                                 buf.at[pl.ds(j * 128, 128)], gsem)
                for j in range(KSUB)]

    cps = gather_chunk(0, bufs[0])
    for cidx in range(EMB_CHUNKS):
        for cp in cps:
            cp.wait()
        if cidx + 1 < EMB_CHUNKS:
            nxt = gather_chunk(cidx + 1, bufs[(cidx + 1) % 2])
        else:
            nxt = []
        b128 = base128 + cidx * KSUB
        pltpu.async_copy(bufs[cidx % 2],
                         eout_hbm.at[pl.ds(b128 * 128, CHUNK)], wsem).wait()
        cps = nxt

  return _emb_kernel


def _node_body(x_ref, emb_ref, W1_ref, b1_ref, W2_ref, b2_ref,
               Wd_ref, bd_ref, wnc_ref, bnc_ref, wne_ref, bne_ref,
               We_ref, be_ref, h_ref, he_ref):
    dot = functools.partial(jnp.dot, preferred_element_type=jnp.float32)
    xb = x_ref[...]
    n1 = float(NUM_NODE * COORD)
    mean1 = jnp.sum(xb) / n1
    xc = xb - mean1
    var1 = jnp.sum(xc * xc) / n1
    hb = xc * lax.rsqrt(var1 + EPS) * wnc_ref[...] + bnc_ref[...]
    h1 = jnp.maximum(dot(hb, W1_ref[...]) + b1_ref[...], 0.0)
    h2 = jnp.maximum(dot(h1, W2_ref[...]) + b2_ref[...], 0.0)
    h3 = jnp.maximum(dot(h2, Wd_ref[...]) + bd_ref[...], 0.0)
    t = h3 + emb_ref[...]
    n2 = float(NUM_NODE * NODE_DIM)
    mean2 = jnp.sum(t) / n2
    tc2 = t - mean2
    var2 = jnp.sum(tc2 * tc2) / n2
    hn = tc2 * lax.rsqrt(var2 + EPS) * wne_ref[...] + bne_ref[...]
    h_ref[...] = hn
    he_ref[...] = jnp.maximum(dot(hn, We_ref[...]) + be_ref[...], 0.0)


def _node_stage(x, emb, W1, b1, W2, b2, Wd, bd, w_nc, b_nc,
                w_ne, b_ne, We, be):
    def full(a):
        return pl.BlockSpec(a.shape, lambda b_: tuple(0 for _ in a.shape))

    grid_specs = [
        pl.BlockSpec((NUM_NODE, COORD), lambda b_: (b_, 0)),
        pl.BlockSpec((NUM_NODE, NODE_DIM), lambda b_: (b_, 0)),
    ] + [full(a) for a in (W1, b1, W2, b2, Wd, bd, w_nc, b_nc, w_ne, b_ne, We, be)]
    return pl.pallas_call(
        _node_body,
        grid=(B,),
        in_specs=grid_specs,
        out_specs=[
            pl.BlockSpec((NUM_NODE, NODE_DIM), lambda b_: (b_, 0)),
            pl.BlockSpec((NUM_NODE, EDGE_DIM), lambda b_: (b_, 0)),
        ],
        out_shape=[
            jax.ShapeDtypeStruct((N_NODES, NODE_DIM), jnp.float32),
            jax.ShapeDtypeStruct((N_NODES, EDGE_DIM), jnp.float32),
        ],
    )(x, emb, W1, b1, W2, b2, Wd, bd, w_nc, b_nc, w_ne, b_ne, We, be)


@functools.cache
def _build_edge_kernel():
  @functools.partial(
      pl.kernel,
      out_type=(jax.ShapeDtypeStruct((MROWS, 128), jnp.float32),
                jax.ShapeDtypeStruct((2 * NW, EDGE_DIM), jnp.float32)),
      mesh=_mesh(),
      compiler_params=pltpu.CompilerParams(use_tc_tiling_on_sc=False),
      scratch_types=[
          pltpu.VMEM((KSUB, 128), jnp.int32),
          pltpu.VMEM((KSUB, 128), jnp.int32),
          pltpu.VMEM((KSUB, 128), jnp.int32),
          pltpu.VMEM((KSUB, 128), jnp.int32),
          pltpu.VMEM((CHUNK, EDGE_DIM), jnp.float32),
          pltpu.VMEM((CHUNK, EDGE_DIM), jnp.float32),
          pltpu.VMEM((CHUNK, EDGE_DIM), jnp.float32),
          pltpu.VMEM((CHUNK, EDGE_DIM), jnp.float32),
          pltpu.VMEM((CHUNK // 8, 128), jnp.float32),
          pltpu.VMEM((CHUNK // 8, 128), jnp.float32),
          pltpu.VMEM((2, EDGE_DIM), jnp.float32),
          pltpu.SemaphoreType.DMA,
          pltpu.SemaphoreType.DMA,
      ],
  )
  def _edge_kernel(row_hbm, col_hbm, he_hbm, m_hbm, stats_hbm,
                   idxr0, idxc0, idxr1, idxc1, rbuf0, cbuf0, rbuf1, cbuf1,
                   wbuf0, wbuf1, sbuf, gsem, wsem):
    w = lax.axis_index("s") * NC + lax.axis_index("c")
    g = w // 2
    hf = w % 2
    main128 = g * (EPB // 128) + hf * (HALF_MAIN // 128)
    self128 = (E // 128) + g * (NUM_NODE // 128) + hf * (HALF_SELF // 128)
    TOTAL = MAIN_CHUNKS + SELF_CHUNKS

    def base_of(k):
        return jnp.where(k < MAIN_CHUNKS, main128 + k * KSUB,
                         self128 + (k - MAIN_CHUNKS) * KSUB)

    bufs = ((idxr0, idxc0, rbuf0, cbuf0, wbuf0),
            (idxr1, idxc1, rbuf1, cbuf1, wbuf1))

    def issue(k, p):
        idxr, idxc, rbuf, cbuf, _ = bufs[p]
        b128 = base_of(k)
        pltpu.sync_copy(row_hbm.at[pl.ds(b128, KSUB)], idxr)
        pltpu.sync_copy(col_hbm.at[pl.ds(b128, KSUB)], idxc)
        for j in range(KSUB):
            pltpu.async_copy(he_hbm.at[idxr.at[j]],
                             rbuf.at[pl.ds(j * 128, 128)], gsem)
            pltpu.async_copy(he_hbm.at[idxc.at[j]],
                             cbuf.at[pl.ds(j * 128, 128)], gsem)

    def drain(p):
        idxr, idxc, rbuf, cbuf, _ = bufs[p]
        for j in range(KSUB):
            pltpu.make_async_copy(he_hbm.at[idxr.at[j]],
                                  rbuf.at[pl.ds(j * 128, 128)], gsem).wait()
            pltpu.make_async_copy(he_hbm.at[idxc.at[j]],
                                  cbuf.at[pl.ds(j * 128, 128)], gsem).wait()

    def compute(k, p, acc):
        _, _, rbuf, cbuf, wbuf = bufs[p]

        def body8(kk, c2):
            s0, q0, s1, q1 = c2
            e0 = kk * 8
            for u in range(8):
                m = (rbuf[e0 + u] + cbuf[e0 + u]) * 0.5
                wbuf[kk, pl.ds(u * 16, 16)] = m
                if u % 2 == 0:
                    s0 = s0 + m
                    q0 = q0 + m * m
                else:
                    s1 = s1 + m
                    q1 = q1 + m * m
            return (s0, q0, s1, q1)

        acc = lax.fori_loop(0, CHUNK // 8, body8, acc)
        pltpu.async_copy(
            wbuf, m_hbm.at[pl.ds(base_of(k) * 16, CHUNK // 8)], wsem).wait()
        return acc

    zero = jnp.zeros((16,), jnp.float32)
    issue(0, 0)

    def pair_body(gi, acc):
        drain(0)
        issue(2 * gi + 1, 1)
        acc = compute(2 * gi, 0, acc)
        drain(1)
        issue(2 * gi + 2, 0)
        acc = compute(2 * gi + 1, 1, acc)
        return acc

    acc = lax.fori_loop(0, (TOTAL - 1) // 2, pair_body,
                        (zero, zero, zero, zero))
    drain(0)
    acc = compute(TOTAL - 1, 0, acc)
    sbuf[0] = acc[0] + acc[2]
    sbuf[1] = acc[1] + acc[3]
    pltpu.sync_copy(sbuf.at[0], stats_hbm.at[w])
    pltpu.sync_copy(sbuf.at[1], stats_hbm.at[NW + w])

  return _edge_kernel


def _final_body(m_ref, stats_ref, Wg128_ref, bg256_ref, wen128_ref, ben128_ref,
                wgn_ref, bgn_ref, ea_ref, u_ref):
    b = pl.program_id(0)
    c = pl.program_id(1)
    stats = stats_ref[...]
    rid = lax.broadcasted_iota(jnp.int32, (2 * NW, EDGE_DIM), 0)
    sel_s = (rid // 2 == b) & (rid < NW)
    sel_q = (rid >= NW) & ((rid - NW) // 2 == b)
    S = jnp.sum(jnp.where(sel_s, stats, 0.0))
    Q = jnp.sum(jnp.where(sel_q, stats, 0.0))
    nrm = float(CNT_E * EDGE_DIM)
    mean = S / nrm
    var = Q / nrm - mean * mean
    inv = lax.rsqrt(var + EPS)
    ea = (m_ref[...] - mean) * inv * wen128_ref[...] + ben128_ref[...]
    ea_ref[...] = ea
    g = jnp.maximum(
        jnp.dot(ea, Wg128_ref[...], preferred_element_type=jnp.float32)
        + bg256_ref[...], 0.0)
    psum = jnp.sum(g, axis=0, keepdims=True)[None]

    @pl.when(c == 0)
    def _():
        u_ref[...] = psum

    @pl.when(c != 0)
    def _():
        u_ref[...] = u_ref[...] + psum

    @pl.when(c == B)
    def _():
        acc = u_ref[...] / float(CNT_E)
        tot = acc[:, :, 0:GLOB_DIM]
        for k in range(1, 8):
            tot = tot + acc[:, :, k * GLOB_DIM:(k + 1) * GLOB_DIM]
        mu = jnp.sum(tot) / float(GLOB_DIM)
        d = tot - mu
        varu = jnp.sum(d * d) / float(GLOB_DIM)
        fin = (d * lax.rsqrt(varu + EPS) * wgn_ref[...][None]
               + bgn_ref[...][None])
        u_ref[...] = jnp.concatenate(
            [fin, jnp.zeros((1, 1, EA_LANES - GLOB_DIM), jnp.float32)], axis=-1)


def _final_stage(m, stats, Wg128, bg256, wen128, ben128, w_gn, b_gn):
    def full(a):
        return pl.BlockSpec(a.shape, lambda b_, c_: tuple(0 for _ in a.shape))

    def edge_map(b_, c_):
        return (jnp.where(c_ < B, b_ * B + c_, B * B + b_), 0)

    rows_per_block = NUM_NODE * EDGE_DIM // 128
    return pl.pallas_call(
        _final_body,
        grid=(B, B + 1),
        in_specs=[pl.BlockSpec((rows_per_block, 128), edge_map),
                  full(stats), full(Wg128), full(bg256), full(wen128),
                  full(ben128), full(w_gn), full(b_gn)],
        out_specs=[
            pl.BlockSpec((rows_per_block, 128), edge_map),
            pl.BlockSpec((1, 1, EA_LANES), lambda b_, c_: (b_, 0, 0)),
        ],
        out_shape=[
            jax.ShapeDtypeStruct((MROWS, 128), jnp.float32),
            jax.ShapeDtypeStruct((B, 1, EA_LANES), jnp.float32),
        ],
    )(m, stats, Wg128, bg256, wen128, ben128, w_gn, b_gn)


def kernel(x, atom_ids, aa_ids, edge_index, W1, b1, W2, b2, Wd, bd,
           atom_emb, aa_emb, w_nc, b_nc, w_ne, b_ne, We, be, w_en, b_en,
           Wg, bg, w_gn, b_gn):
    loops = jnp.arange(N_NODES, dtype=edge_index.dtype)
    row = jnp.concatenate([edge_index[0], loops])
    col = jnp.concatenate([edge_index[1], loops])
    ei = jnp.stack([row, col])
    row128 = row.astype(jnp.int32).reshape(ETOT // 128, 128)
    col128 = col.astype(jnp.int32).reshape(ETOT // 128, 128)
    cid128 = (atom_ids.astype(jnp.int32) * 32
              + aa_ids.astype(jnp.int32)).reshape(N_NODES // 128, 128)
    comb_tbl = (atom_emb.astype(jnp.float32)[:, None, :]
                + aa_emb.astype(jnp.float32)[None, :, :]).reshape(-1, NODE_DIM)

    r2 = lambda a: a.reshape(1, -1).astype(jnp.float32)
    emb = _build_emb_kernel()(cid128, comb_tbl)
    h, he = _node_stage(x, emb, W1, r2(b1), W2, r2(b2), Wd, r2(bd),
                        r2(w_nc), r2(b_nc), r2(w_ne), r2(b_ne), We, r2(be))
    m, stats = _build_edge_kernel()(row128, col128, he)
    Wg128 = jnp.kron(jnp.eye(8, dtype=jnp.float32), Wg.astype(jnp.float32))
    ea_d, u3 = _final_stage(m, stats, Wg128, r2(jnp.tile(bg, 8)),
                            r2(jnp.tile(w_en, 8)), r2(jnp.tile(b_en, 8)),
                            r2(w_gn), r2(b_gn))
    edge_attr = ea_d.reshape(ETOT, EDGE_DIM)
    return (h, edge_attr, u3[:, 0, :GLOB_DIM], ei)


# speedup vs baseline: 1.6291x; 1.2580x over previous
import functools

import jax
import jax.numpy as jnp
from jax import lax
from jax.experimental import pallas as pl
from jax.experimental.pallas import tpu as pltpu
from jax.experimental.pallas import tpu_sc as plsc

N_NODES = 98304
B = 16
NUM_NODE = 6144
E = 1572864
ETOT = E + N_NODES
COORD = 3
NODE_DIM = 48
EDGE_DIM = 16
GLOB_DIM = 32
EPS = 1e-5

EPB = E // B
CNT_E = EPB + NUM_NODE

NC, NS = 2, 16
NW = NC * NS
HALF_MAIN = EPB // 2
HALF_SELF = NUM_NODE // 2
CHUNK = 1024
KSUB = CHUNK // 128
MAIN_CHUNKS = HALF_MAIN // CHUNK
SELF_CHUNKS = HALF_SELF // CHUNK
NODES_PW = N_NODES // NW
EMB_CHUNKS = NODES_PW // CHUNK

MROWS = ETOT * EDGE_DIM // 128
EA_LANES = 8 * GLOB_DIM


@functools.cache
def _mesh():
    return plsc.VectorSubcoreMesh(
        core_axis_name="c", subcore_axis_name="s", num_cores=NC, num_subcores=NS)


@functools.cache
def _build_emb_kernel():
  @functools.partial(
      pl.kernel,
      out_type=jax.ShapeDtypeStruct((N_NODES, NODE_DIM), jnp.float32),
      mesh=_mesh(),
      compiler_params=pltpu.CompilerParams(use_tc_tiling_on_sc=False),
      scratch_types=[
          pltpu.VMEM((KSUB, 128), jnp.int32),
          pltpu.VMEM((CHUNK, NODE_DIM), jnp.float32),
          pltpu.VMEM((CHUNK, NODE_DIM), jnp.float32),
          pltpu.SemaphoreType.DMA,
          pltpu.SemaphoreType.DMA,
      ],
  )
  def _emb_kernel(cid_hbm, tbl_hbm, eout_hbm, idc, ebuf0, ebuf1, gsem, wsem):
    w = lax.axis_index("s") * NC + lax.axis_index("c")
    base128 = w * (NODES_PW // 128)
    bufs = (ebuf0, ebuf1)

    def gather_chunk(cidx, buf):
        b128 = base128 + cidx * KSUB
        pltpu.sync_copy(cid_hbm.at[pl.ds(b128, KSUB)], idc)
        return [pltpu.async_copy(tbl_hbm.at[idc.at[j]],
                                 buf.at[pl.ds(j * 128, 128)], gsem)
                for j in range(KSUB)]

    cps = gather_chunk(0, bufs[0])
    for cidx in range(EMB_CHUNKS):
        for cp in cps:
            cp.wait()
        if cidx + 1 < EMB_CHUNKS:
            nxt = gather_chunk(cidx + 1, bufs[(cidx + 1) % 2])
        else:
            nxt = []
        b128 = base128 + cidx * KSUB
        pltpu.async_copy(bufs[cidx % 2],
                         eout_hbm.at[pl.ds(b128 * 128, CHUNK)], wsem).wait()
        cps = nxt

  return _emb_kernel


def _node_body(x_ref, emb_ref, W1_ref, b1_ref, W2_ref, b2_ref,
               Wd_ref, bd_ref, wnc_ref, bnc_ref, wne_ref, bne_ref,
               We_ref, be_ref, h_ref, he_ref):
    dot = functools.partial(jnp.dot, preferred_element_type=jnp.float32)
    xb = x_ref[...]
    n1 = float(NUM_NODE * COORD)
    mean1 = jnp.sum(xb) / n1
    xc = xb - mean1
    var1 = jnp.sum(xc * xc) / n1
    hb = xc * lax.rsqrt(var1 + EPS) * wnc_ref[...] + bnc_ref[...]
    h1 = jnp.maximum(dot(hb, W1_ref[...]) + b1_ref[...], 0.0)
    h2 = jnp.maximum(dot(h1, W2_ref[...]) + b2_ref[...], 0.0)
    h3 = jnp.maximum(dot(h2, Wd_ref[...]) + bd_ref[...], 0.0)
    t = h3 + emb_ref[...]
    n2 = float(NUM_NODE * NODE_DIM)
    mean2 = jnp.sum(t) / n2
    tc2 = t - mean2
    var2 = jnp.sum(tc2 * tc2) / n2
    hn = tc2 * lax.rsqrt(var2 + EPS) * wne_ref[...] + bne_ref[...]
    h_ref[...] = hn
    he_ref[...] = jnp.maximum(dot(hn, We_ref[...]) + be_ref[...], 0.0)


def _node_stage(x, emb, W1, b1, W2, b2, Wd, bd, w_nc, b_nc,
                w_ne, b_ne, We, be):
    def full(a):
        return pl.BlockSpec(a.shape, lambda b_: tuple(0 for _ in a.shape))

    grid_specs = [
        pl.BlockSpec((NUM_NODE, COORD), lambda b_: (b_, 0)),
        pl.BlockSpec((NUM_NODE, NODE_DIM), lambda b_: (b_, 0)),
    ] + [full(a) for a in (W1, b1, W2, b2, Wd, bd, w_nc, b_nc, w_ne, b_ne, We, be)]
    return pl.pallas_call(
        _node_body,
        grid=(B,),
        in_specs=grid_specs,
        out_specs=[
            pl.BlockSpec((NUM_NODE, NODE_DIM), lambda b_: (b_, 0)),
            pl.BlockSpec((NUM_NODE, EDGE_DIM), lambda b_: (b_, 0)),
        ],
        out_shape=[
            jax.ShapeDtypeStruct((N_NODES, NODE_DIM), jnp.float32),
            jax.ShapeDtypeStruct((N_NODES, EDGE_DIM), jnp.float32),
        ],
    )(x, emb, W1, b1, W2, b2, Wd, bd, w_nc, b_nc, w_ne, b_ne, We, be)


@functools.cache
def _build_edge_kernel():
  @functools.partial(
      pl.kernel,
      out_type=(jax.ShapeDtypeStruct((MROWS, 128), jnp.float32),
                jax.ShapeDtypeStruct((2 * NW, EDGE_DIM), jnp.float32)),
      mesh=_mesh(),
      compiler_params=pltpu.CompilerParams(use_tc_tiling_on_sc=False),
      scratch_types=[
          pltpu.VMEM((KSUB, 128), jnp.int32),
          pltpu.VMEM((KSUB, 128), jnp.int32),
          pltpu.VMEM((KSUB, 128), jnp.int32),
          pltpu.VMEM((KSUB, 128), jnp.int32),
          pltpu.VMEM((CHUNK, EDGE_DIM), jnp.float32),
          pltpu.VMEM((CHUNK, EDGE_DIM), jnp.float32),
          pltpu.VMEM((CHUNK, EDGE_DIM), jnp.float32),
          pltpu.VMEM((CHUNK, EDGE_DIM), jnp.float32),
          pltpu.VMEM((CHUNK // 8, 128), jnp.float32),
          pltpu.VMEM((CHUNK // 8, 128), jnp.float32),
          pltpu.VMEM((2, EDGE_DIM), jnp.float32),
          pltpu.SemaphoreType.DMA,
          pltpu.SemaphoreType.DMA,
      ],
  )
  def _edge_kernel(row_hbm, col_hbm, he_hbm, m_hbm, stats_hbm,
                   idxr0, idxc0, idxr1, idxc1, rbuf0, cbuf0, rbuf1, cbuf1,
                   wbuf0, wbuf1, sbuf, gsem, wsem):
    w = lax.axis_index("s") * NC + lax.axis_index("c")
    g = w // 2
    hf = w % 2
    main128 = g * (EPB // 128) + hf * (HALF_MAIN // 128)
    self128 = (E // 128) + g * (NUM_NODE // 128) + hf * (HALF_SELF // 128)
    TOTAL = MAIN_CHUNKS + SELF_CHUNKS

    def base_of(k):
        return jnp.where(k < MAIN_CHUNKS, main128 + k * KSUB,
                         self128 + (k - MAIN_CHUNKS) * KSUB)

    bufs = ((idxr0, idxc0, rbuf0, cbuf0, wbuf0),
            (idxr1, idxc1, rbuf1, cbuf1, wbuf1))

    def issue(k, p):
        idxr, idxc, rbuf, cbuf, _ = bufs[p]
        b128 = base_of(k)
        pltpu.sync_copy(row_hbm.at[pl.ds(b128, KSUB)], idxr)
        pltpu.sync_copy(col_hbm.at[pl.ds(b128, KSUB)], idxc)
        for j in range(KSUB):
            pltpu.async_copy(he_hbm.at[idxr.at[j]],
                             rbuf.at[pl.ds(j * 128, 128)], gsem)
            pltpu.async_copy(he_hbm.at[idxc.at[j]],
                             cbuf.at[pl.ds(j * 128, 128)], gsem)

    def drain(p):
        idxr, idxc, rbuf, cbuf, _ = bufs[p]
        for j in range(KSUB):
            pltpu.make_async_copy(he_hbm.at[idxr.at[j]],
                                  rbuf.at[pl.ds(j * 128, 128)], gsem).wait()
            pltpu.make_async_copy(he_hbm.at[idxc.at[j]],
                                  cbuf.at[pl.ds(j * 128, 128)], gsem).wait()

    def compute(k, p, acc):
        _, _, rbuf, cbuf, wbuf = bufs[p]

        def body8(kk, c2):
            s0, q0, s1, q1 = c2
            e0 = kk * 8
            for u in range(8):
                m = (rbuf[e0 + u] + cbuf[e0 + u]) * 0.5
                wbuf[kk, pl.ds(u * 16, 16)] = m
                if u % 2 == 0:
                    s0 = s0 + m
                    q0 = q0 + m * m
                else:
                    s1 = s1 + m
                    q1 = q1 + m * m
            return (s0, q0, s1, q1)

        acc = lax.fori_loop(0, CHUNK // 8, body8, acc)
        pltpu.async_copy(
            wbuf, m_hbm.at[pl.ds(base_of(k) * 16, CHUNK // 8)], wsem).wait()
        return acc

    zero = jnp.zeros((16,), jnp.float32)
    issue(0, 0)

    def pair_body(gi, acc):
        drain(0)
        issue(2 * gi + 1, 1)
        acc = compute(2 * gi, 0, acc)
        drain(1)
        issue(2 * gi + 2, 0)
        acc = compute(2 * gi + 1, 1, acc)
        return acc

    acc = lax.fori_loop(0, (TOTAL - 1) // 2, pair_body,
                        (zero, zero, zero, zero))
    drain(0)
    acc = compute(TOTAL - 1, 0, acc)
    sbuf[0] = acc[0] + acc[2]
    sbuf[1] = acc[1] + acc[3]
    pltpu.sync_copy(sbuf.at[0], stats_hbm.at[w])
    pltpu.sync_copy(sbuf.at[1], stats_hbm.at[NW + w])

  return _edge_kernel


def _final_body(m_ref, stats_ref, Wg128_ref, bg256_ref, wen128_ref, ben128_ref,
                wgn_ref, bgn_ref, ea_ref, u_ref):
    b = pl.program_id(0)
    c = pl.program_id(1)
    stats = stats_ref[...]
    rid = lax.broadcasted_iota(jnp.int32, (2 * NW, EDGE_DIM), 0)
    sel_s = (rid // 2 == b) & (rid < NW)
    sel_q = (rid >= NW) & ((rid - NW) // 2 == b)
    S = jnp.sum(jnp.where(sel_s, stats, 0.0))
    Q = jnp.sum(jnp.where(sel_q, stats, 0.0))
    nrm = float(CNT_E * EDGE_DIM)
    mean = S / nrm
    var = Q / nrm - mean * mean
    inv = lax.rsqrt(var + EPS)
    ea = (m_ref[...] - mean) * inv * wen128_ref[...] + ben128_ref[...]
    ea_ref[...] = ea
    g = jnp.maximum(
        jnp.dot(ea, Wg128_ref[...], preferred_element_type=jnp.float32)
        + bg256_ref[...], 0.0)
    psum = jnp.sum(g, axis=0, keepdims=True)[None]

    @pl.when(c == 0)
    def _():
        u_ref[...] = psum

    @pl.when(c != 0)
    def _():
        u_ref[...] = u_ref[...] + psum

    @pl.when(c == B)
    def _():
        acc = u_ref[...] / float(CNT_E)
        tot = acc[:, :, 0:GLOB_DIM]
        for k in range(1, 8):
            tot = tot + acc[:, :, k * GLOB_DIM:(k + 1) * GLOB_DIM]
        mu = jnp.sum(tot) / float(GLOB_DIM)
        d = tot - mu
        varu = jnp.sum(d * d) / float(GLOB_DIM)
        fin = (d * lax.rsqrt(varu + EPS) * wgn_ref[...][None]
               + bgn_ref[...][None])
        u_ref[...] = jnp.concatenate(
            [fin, jnp.zeros((1, 1, EA_LANES - GLOB_DIM), jnp.float32)], axis=-1)


def _final_stage(m, stats, Wg128, bg256, wen128, ben128, w_gn, b_gn):
    def full(a):
        return pl.BlockSpec(a.shape, lambda b_, c_: tuple(0 for _ in a.shape))

    def edge_map(b_, c_):
        return (jnp.where(c_ < B, b_ * B + c_, B * B + b_), 0)

    rows_per_block = NUM_NODE * EDGE_DIM // 128
    return pl.pallas_call(
        _final_body,
        grid=(B, B + 1),
        in_specs=[pl.BlockSpec((rows_per_block, 128), edge_map),
                  full(stats), full(Wg128), full(bg256), full(wen128),
                  full(ben128), full(w_gn), full(b_gn)],
        out_specs=[
            pl.BlockSpec((rows_per_block, 128), edge_map),
            pl.BlockSpec((1, 1, EA_LANES), lambda b_, c_: (b_, 0, 0)),
        ],
        out_shape=[
            jax.ShapeDtypeStruct((MROWS, 128), jnp.float32),
            jax.ShapeDtypeStruct((B, 1, EA_LANES), jnp.float32),
        ],
    )(m, stats, Wg128, bg256, wen128, ben128, w_gn, b_gn)


def kernel(x, atom_ids, aa_ids, edge_index, W1, b1, W2, b2, Wd, bd,
           atom_emb, aa_emb, w_nc, b_nc, w_ne, b_ne, We, be, w_en, b_en,
           Wg, bg, w_gn, b_gn):
    loops = jnp.arange(N_NODES, dtype=edge_index.dtype)
    row = jnp.concatenate([edge_index[0], loops])
    col = jnp.concatenate([edge_index[1], loops])
    ei = jnp.stack([row, col])
    row128 = row.astype(jnp.int32).reshape(ETOT // 128, 128)
    col128 = col.astype(jnp.int32).reshape(ETOT // 128, 128)
    cid128 = (atom_ids.astype(jnp.int32) * 32
              + aa_ids.astype(jnp.int32)).reshape(N_NODES // 128, 128)
    comb_tbl = (atom_emb.astype(jnp.float32)[:, None, :]
                + aa_emb.astype(jnp.float32)[None, :, :]).reshape(-1, NODE_DIM)

    r2 = lambda a: a.reshape(1, -1).astype(jnp.float32)
    emb = _build_emb_kernel()(cid128, comb_tbl)
    h, he = _node_stage(x, emb, W1, r2(b1), W2, r2(b2), Wd, r2(bd),
                        r2(w_nc), r2(b_nc), r2(w_ne), r2(b_ne), We, r2(be))
    m, stats = _build_edge_kernel()(row128, col128, he)
    Wg128 = jnp.kron(jnp.eye(8, dtype=jnp.float32), Wg.astype(jnp.float32))
    ea_d, u3 = _final_stage(m, stats, Wg128, r2(jnp.tile(bg, 8)),
                            r2(jnp.tile(w_en, 8)), r2(jnp.tile(b_en, 8)),
                            r2(w_gn), r2(b_gn))
    edge_attr = (ea_d.reshape(MROWS, 8, EDGE_DIM)
                 .transpose(2, 0, 1).reshape(EDGE_DIM, ETOT).T)
    return (h, edge_attr, u3[:, 0, :GLOB_DIM], ei)


# speedup vs baseline: 1.6608x; 1.0194x over previous
import functools

import jax
import jax.numpy as jnp
from jax import lax
from jax.experimental import pallas as pl
from jax.experimental.pallas import tpu as pltpu
from jax.experimental.pallas import tpu_sc as plsc

N_NODES = 98304
B = 16
NUM_NODE = 6144
E = 1572864
ETOT = E + N_NODES
COORD = 3
NODE_DIM = 48
EDGE_DIM = 16
GLOB_DIM = 32
EPS = 1e-5

EPB = E // B
CNT_E = EPB + NUM_NODE

NC, NS = 2, 16
NW = NC * NS
HALF_MAIN = EPB // 2
HALF_SELF = NUM_NODE // 2
CHUNK = 1024
KSUB = CHUNK // 128
MAIN_CHUNKS = HALF_MAIN // CHUNK
SELF_CHUNKS = HALF_SELF // CHUNK
NODES_PW = N_NODES // NW
EMB_CHUNKS = NODES_PW // CHUNK

MROWS = ETOT * EDGE_DIM // 128
EA_LANES = 8 * GLOB_DIM


@functools.cache
def _mesh():
    return plsc.VectorSubcoreMesh(
        core_axis_name="c", subcore_axis_name="s", num_cores=NC, num_subcores=NS)


@functools.cache
def _build_emb_kernel():
  @functools.partial(
      pl.kernel,
      out_type=jax.ShapeDtypeStruct((N_NODES, NODE_DIM), jnp.float32),
      mesh=_mesh(),
      compiler_params=pltpu.CompilerParams(use_tc_tiling_on_sc=False),
      scratch_types=[
          pltpu.VMEM((KSUB, 128), jnp.int32),
          pltpu.VMEM((CHUNK, NODE_DIM), jnp.float32),
          pltpu.VMEM((CHUNK, NODE_DIM), jnp.float32),
          pltpu.SemaphoreType.DMA,
          pltpu.SemaphoreType.DMA,
      ],
  )
  def _emb_kernel(cid_hbm, tbl_hbm, eout_hbm, idc, ebuf0, ebuf1, gsem, wsem):
    w = lax.axis_index("s") * NC + lax.axis_index("c")
    base128 = w * (NODES_PW // 128)
    bufs = (ebuf0, ebuf1)

    def gather_chunk(cidx, buf):
        b128 = base128 + cidx * KSUB
        pltpu.sync_copy(cid_hbm.at[pl.ds(b128, KSUB)], idc)
        return [pltpu.async_copy(tbl_hbm.at[idc.at[j]],
                                 buf.at[pl.ds(j * 128, 128)], gsem)
                for j in range(KSUB)]

    cps = gather_chunk(0, bufs[0])
    for cidx in range(EMB_CHUNKS):
        for cp in cps:
            cp.wait()
        if cidx + 1 < EMB_CHUNKS:
            nxt = gather_chunk(cidx + 1, bufs[(cidx + 1) % 2])
        else:
            nxt = []
        b128 = base128 + cidx * KSUB
        pltpu.async_copy(bufs[cidx % 2],
                         eout_hbm.at[pl.ds(b128 * 128, CHUNK)], wsem).wait()
        cps = nxt

  return _emb_kernel


def _node_body(x_ref, emb_ref, W1_ref, b1_ref, W2_ref, b2_ref,
               Wd_ref, bd_ref, wnc_ref, bnc_ref, wne_ref, bne_ref,
               We_ref, be_ref, h_ref, he_ref):
    dot = functools.partial(jnp.dot, preferred_element_type=jnp.float32)
    xb = x_ref[...]
    n1 = float(NUM_NODE * COORD)
    mean1 = jnp.sum(xb) / n1
    xc = xb - mean1
    var1 = jnp.sum(xc * xc) / n1
    hb = xc * lax.rsqrt(var1 + EPS) * wnc_ref[...] + bnc_ref[...]
    h1 = jnp.maximum(dot(hb, W1_ref[...]) + b1_ref[...], 0.0)
    h2 = jnp.maximum(dot(h1, W2_ref[...]) + b2_ref[...], 0.0)
    h3 = jnp.maximum(dot(h2, Wd_ref[...]) + bd_ref[...], 0.0)
    t = h3 + emb_ref[...]
    n2 = float(NUM_NODE * NODE_DIM)
    mean2 = jnp.sum(t) / n2
    tc2 = t - mean2
    var2 = jnp.sum(tc2 * tc2) / n2
    hn = tc2 * lax.rsqrt(var2 + EPS) * wne_ref[...] + bne_ref[...]
    h_ref[...] = hn.T
    he_ref[...] = jnp.maximum(dot(hn, We_ref[...]) + be_ref[...], 0.0)


def _node_stage(x, emb, W1, b1, W2, b2, Wd, bd, w_nc, b_nc,
                w_ne, b_ne, We, be):
    def full(a):
        return pl.BlockSpec(a.shape, lambda b_: tuple(0 for _ in a.shape))

    grid_specs = [
        pl.BlockSpec((NUM_NODE, COORD), lambda b_: (b_, 0)),
        pl.BlockSpec((NUM_NODE, NODE_DIM), lambda b_: (b_, 0)),
    ] + [full(a) for a in (W1, b1, W2, b2, Wd, bd, w_nc, b_nc, w_ne, b_ne, We, be)]
    return pl.pallas_call(
        _node_body,
        grid=(B,),
        in_specs=grid_specs,
        out_specs=[
            pl.BlockSpec((NODE_DIM, NUM_NODE), lambda b_: (0, b_)),
            pl.BlockSpec((NUM_NODE, EDGE_DIM), lambda b_: (b_, 0)),
        ],
        out_shape=[
            jax.ShapeDtypeStruct((NODE_DIM, N_NODES), jnp.float32),
            jax.ShapeDtypeStruct((N_NODES, EDGE_DIM), jnp.float32),
        ],
    )(x, emb, W1, b1, W2, b2, Wd, bd, w_nc, b_nc, w_ne, b_ne, We, be)


@functools.cache
def _build_edge_kernel():
  @functools.partial(
      pl.kernel,
      out_type=(jax.ShapeDtypeStruct((MROWS, 128), jnp.float32),
                jax.ShapeDtypeStruct((2 * NW, EDGE_DIM), jnp.float32)),
      mesh=_mesh(),
      compiler_params=pltpu.CompilerParams(use_tc_tiling_on_sc=False),
      scratch_types=[
          pltpu.VMEM((KSUB, 128), jnp.int32),
          pltpu.VMEM((KSUB, 128), jnp.int32),
          pltpu.VMEM((KSUB, 128), jnp.int32),
          pltpu.VMEM((KSUB, 128), jnp.int32),
          pltpu.VMEM((CHUNK, EDGE_DIM), jnp.float32),
          pltpu.VMEM((CHUNK, EDGE_DIM), jnp.float32),
          pltpu.VMEM((CHUNK, EDGE_DIM), jnp.float32),
          pltpu.VMEM((CHUNK, EDGE_DIM), jnp.float32),
          pltpu.VMEM((CHUNK // 8, 128), jnp.float32),
          pltpu.VMEM((CHUNK // 8, 128), jnp.float32),
          pltpu.VMEM((2, EDGE_DIM), jnp.float32),
          pltpu.SemaphoreType.DMA,
          pltpu.SemaphoreType.DMA,
      ],
  )
  def _edge_kernel(row_hbm, col_hbm, he_hbm, m_hbm, stats_hbm,
                   idxr0, idxc0, idxr1, idxc1, rbuf0, cbuf0, rbuf1, cbuf1,
                   wbuf0, wbuf1, sbuf, gsem, wsem):
    w = lax.axis_index("s") * NC + lax.axis_index("c")
    g = w // 2
    hf = w % 2
    main128 = g * (EPB // 128) + hf * (HALF_MAIN // 128)
    self128 = (E // 128) + g * (NUM_NODE // 128) + hf * (HALF_SELF // 128)
    TOTAL = MAIN_CHUNKS + SELF_CHUNKS

    def base_of(k):
        return jnp.where(k < MAIN_CHUNKS, main128 + k * KSUB,
                         self128 + (k - MAIN_CHUNKS) * KSUB)

    bufs = ((idxr0, idxc0, rbuf0, cbuf0, wbuf0),
            (idxr1, idxc1, rbuf1, cbuf1, wbuf1))

    def issue(k, p):
        idxr, idxc, rbuf, cbuf, _ = bufs[p]
        b128 = base_of(k)
        pltpu.sync_copy(row_hbm.at[pl.ds(b128, KSUB)], idxr)
        pltpu.sync_copy(col_hbm.at[pl.ds(b128, KSUB)], idxc)
        for j in range(KSUB):
            pltpu.async_copy(he_hbm.at[idxr.at[j]],
                             rbuf.at[pl.ds(j * 128, 128)], gsem)
            pltpu.async_copy(he_hbm.at[idxc.at[j]],
                             cbuf.at[pl.ds(j * 128, 128)], gsem)

    def drain(p):
        idxr, idxc, rbuf, cbuf, _ = bufs[p]
        for j in range(KSUB):
            pltpu.make_async_copy(he_hbm.at[idxr.at[j]],
                                  rbuf.at[pl.ds(j * 128, 128)], gsem).wait()
            pltpu.make_async_copy(he_hbm.at[idxc.at[j]],
                                  cbuf.at[pl.ds(j * 128, 128)], gsem).wait()

    def compute(k, p, acc):
        _, _, rbuf, cbuf, wbuf = bufs[p]

        def body8(kk, c2):
            s0, q0, s1, q1 = c2
            e0 = kk * 8
            for u in range(8):
                m = (rbuf[e0 + u] + cbuf[e0 + u]) * 0.5
                wbuf[kk, pl.ds(u * 16, 16)] = m
                if u % 2 == 0:
                    s0 = s0 + m
                    q0 = q0 + m * m
                else:
                    s1 = s1 + m
                    q1 = q1 + m * m
            return (s0, q0, s1, q1)

        acc = lax.fori_loop(0, CHUNK // 8, body8, acc)
        pltpu.async_copy(
            wbuf, m_hbm.at[pl.ds(base_of(k) * 16, CHUNK // 8)], wsem).wait()
        return acc

    zero = jnp.zeros((16,), jnp.float32)
    issue(0, 0)

    def pair_body(gi, acc):
        drain(0)
        issue(2 * gi + 1, 1)
        acc = compute(2 * gi, 0, acc)
        drain(1)
        issue(2 * gi + 2, 0)
        acc = compute(2 * gi + 1, 1, acc)
        return acc

    acc = lax.fori_loop(0, (TOTAL - 1) // 2, pair_body,
                        (zero, zero, zero, zero))
    drain(0)
    acc = compute(TOTAL - 1, 0, acc)
    sbuf[0] = acc[0] + acc[2]
    sbuf[1] = acc[1] + acc[3]
    pltpu.sync_copy(sbuf.at[0], stats_hbm.at[w])
    pltpu.sync_copy(sbuf.at[1], stats_hbm.at[NW + w])

  return _edge_kernel


def _final_body(m_ref, stats_ref, Wg128_ref, bg256_ref, wen128_ref, ben128_ref,
                wgn_ref, bgn_ref, ea_ref, u_ref):
    b = pl.program_id(0)
    c = pl.program_id(1)
    stats = stats_ref[...]
    rid = lax.broadcasted_iota(jnp.int32, (2 * NW, EDGE_DIM), 0)
    sel_s = (rid // 2 == b) & (rid < NW)
    sel_q = (rid >= NW) & ((rid - NW) // 2 == b)
    S = jnp.sum(jnp.where(sel_s, stats, 0.0))
    Q = jnp.sum(jnp.where(sel_q, stats, 0.0))
    nrm = float(CNT_E * EDGE_DIM)
    mean = S / nrm
    var = Q / nrm - mean * mean
    inv = lax.rsqrt(var + EPS)
    ea = (m_ref[...] - mean) * inv * wen128_ref[...] + ben128_ref[...]
    ea_ref[...] = ea
    g = jnp.maximum(
        jnp.dot(ea, Wg128_ref[...], preferred_element_type=jnp.float32)
        + bg256_ref[...], 0.0)
    psum = jnp.sum(g, axis=0, keepdims=True)[None]

    @pl.when(c == 0)
    def _():
        u_ref[...] = psum

    @pl.when(c != 0)
    def _():
        u_ref[...] = u_ref[...] + psum

    @pl.when(c == B)
    def _():
        acc = u_ref[...] / float(CNT_E)
        tot = acc[:, :, 0:GLOB_DIM]
        for k in range(1, 8):
            tot = tot + acc[:, :, k * GLOB_DIM:(k + 1) * GLOB_DIM]
        mu = jnp.sum(tot) / float(GLOB_DIM)
        d = tot - mu
        varu = jnp.sum(d * d) / float(GLOB_DIM)
        fin = (d * lax.rsqrt(varu + EPS) * wgn_ref[...][None]
               + bgn_ref[...][None])
        u_ref[...] = jnp.concatenate(
            [fin, jnp.zeros((1, 1, EA_LANES - GLOB_DIM), jnp.float32)], axis=-1)


def _final_stage(m, stats, Wg128, bg256, wen128, ben128, w_gn, b_gn):
    def full(a):
        return pl.BlockSpec(a.shape, lambda b_, c_: tuple(0 for _ in a.shape))

    def edge_map(b_, c_):
        return (jnp.where(c_ < B, b_ * B + c_, B * B + b_), 0)

    rows_per_block = NUM_NODE * EDGE_DIM // 128
    return pl.pallas_call(
        _final_body,
        grid=(B, B + 1),
        in_specs=[pl.BlockSpec((rows_per_block, 128), edge_map),
                  full(stats), full(Wg128), full(bg256), full(wen128),
                  full(ben128), full(w_gn), full(b_gn)],
        out_specs=[
            pl.BlockSpec((rows_per_block, 128), edge_map),
            pl.BlockSpec((1, 1, EA_LANES), lambda b_, c_: (b_, 0, 0)),
        ],
        out_shape=[
            jax.ShapeDtypeStruct((MROWS, 128), jnp.float32),
            jax.ShapeDtypeStruct((B, 1, EA_LANES), jnp.float32),
        ],
    )(m, stats, Wg128, bg256, wen128, ben128, w_gn, b_gn)


def kernel(x, atom_ids, aa_ids, edge_index, W1, b1, W2, b2, Wd, bd,
           atom_emb, aa_emb, w_nc, b_nc, w_ne, b_ne, We, be, w_en, b_en,
           Wg, bg, w_gn, b_gn):
    loops = jnp.arange(N_NODES, dtype=edge_index.dtype)
    row = jnp.concatenate([edge_index[0], loops])
    col = jnp.concatenate([edge_index[1], loops])
    ei = jnp.stack([row, col])
    row128 = row.astype(jnp.int32).reshape(ETOT // 128, 128)
    col128 = col.astype(jnp.int32).reshape(ETOT // 128, 128)
    cid128 = (atom_ids.astype(jnp.int32) * 32
              + aa_ids.astype(jnp.int32)).reshape(N_NODES // 128, 128)
    comb_tbl = (atom_emb.astype(jnp.float32)[:, None, :]
                + aa_emb.astype(jnp.float32)[None, :, :]).reshape(-1, NODE_DIM)

    r2 = lambda a: a.reshape(1, -1).astype(jnp.float32)
    emb = _build_emb_kernel()(cid128, comb_tbl)
    ht, he = _node_stage(x, emb, W1, r2(b1), W2, r2(b2), Wd, r2(bd),
                         r2(w_nc), r2(b_nc), r2(w_ne), r2(b_ne), We, r2(be))
    h = ht.T
    m, stats = _build_edge_kernel()(row128, col128, he)
    Wg128 = jnp.kron(jnp.eye(8, dtype=jnp.float32), Wg.astype(jnp.float32))
    ea_d, u3 = _final_stage(m, stats, Wg128, r2(jnp.tile(bg, 8)),
                            r2(jnp.tile(w_en, 8)), r2(jnp.tile(b_en, 8)),
                            r2(w_gn), r2(b_gn))
    edge_attr = (ea_d.reshape(MROWS, 8, EDGE_DIM)
                 .transpose(2, 0, 1).reshape(EDGE_DIM, ETOT).T)
    return (h, edge_attr, u3[:, 0, :GLOB_DIM], ei)
